# baseline (speedup 1.0000x reference)
import functools

import jax
import jax.numpy as jnp
from jax.experimental import pallas as pl
from jax.experimental.pallas import tpu as pltpu

F32 = jnp.float32
B = 256
R = 2000
EPS = 1e-5


def _dot(a, b, ca, cb):
    return jax.lax.dot_general(a, b, (((ca,), (cb,)), ((), ())),
                               precision=jax.lax.Precision.HIGHEST,
                               preferred_element_type=F32)


def _onehot(ids):
    return (jax.lax.broadcasted_iota(jnp.int32, (ids.shape[0], B), 1)
            == ids).astype(F32)


def _stage1(x2d, ids3, alpha_r, gamma_r, K):
    def body(x_ref, ids_ref, alpha_ref, gamma_ref, scale_ref, acc_ref):
        i = pl.program_id(0)
        oh = _onehot(ids_ref[0])
        x = x_ref[...]
        vals = jnp.concatenate([x, x * x, jnp.ones((R, 8), F32)], axis=1)
        blk = _dot(oh, vals, 0, 0)

        @pl.when(i == 0)
        def _():
            acc_ref[...] = blk

        @pl.when(i > 0)
        def _():
            acc_ref[...] += blk

        @pl.when(i == K - 1)
        def _():
            acc = acc_ref[...]
            inv = 1.0 / jnp.maximum(acc[:, 64:65], 1.0)
            mean = acc[:, 0:32] * inv
            ex2 = acc[:, 32:64] * inv
            a = alpha_ref[...]
            var = ex2 - (2.0 * a - a * a) * mean * mean
            scale_ref[...] = gamma_ref[...] * jax.lax.rsqrt(var + EPS)

    return pl.pallas_call(
        body,
        grid=(K,),
        in_specs=[
            pl.BlockSpec((R, 32), lambda i: (i, 0)),
            pl.BlockSpec((1, R, 1), lambda i: (i, 0, 0)),
            pl.BlockSpec((1, 32), lambda i: (0, 0)),
            pl.BlockSpec((1, 32), lambda i: (0, 0)),
        ],
        out_specs=pl.BlockSpec((B, 32), lambda i: (0, 0)),
        out_shape=jax.ShapeDtypeStruct((B, 32), F32),
        scratch_shapes=[pltpu.VMEM((B, 72), F32)],
    )(x2d, ids3, alpha_r, gamma_r)


def _stage2(x2d, ids3, scale, lin_wT, K, N):
    def body(x_ref, ids_ref, scale_ref, w_ref, e_ref, s_ref):
        i = pl.program_id(0)
        oh = _onehot(ids_ref[0])
        sc = _dot(oh, scale_ref[...], 1, 0)
        logits = _dot(x_ref[...] * sc, w_ref[...], 1, 0)
        eb = jnp.exp(logits)
        e_ref[...] = eb
        e8 = jnp.concatenate([eb, jnp.zeros((R, 5), F32)], axis=1)
        blk = _dot(oh, e8, 0, 0)

        @pl.when(i == 0)
        def _():
            s_ref[...] = blk

        @pl.when(i > 0)
        def _():
            s_ref[...] += blk

    return pl.pallas_call(
        body,
        grid=(K,),
        in_specs=[
            pl.BlockSpec((R, 32), lambda i: (i, 0)),
            pl.BlockSpec((1, R, 1), lambda i: (i, 0, 0)),
            pl.BlockSpec((B, 32), lambda i: (0, 0)),
            pl.BlockSpec((32, 3), lambda i: (0, 0)),
        ],
        out_specs=[
            pl.BlockSpec((R, 3), lambda i: (i, 0)),
            pl.BlockSpec((B, 8), lambda i: (0, 0)),
        ],
        out_shape=[
            jax.ShapeDtypeStruct((N, 3), F32),
            jax.ShapeDtypeStruct((B, 8), F32),
        ],
    )(x2d, ids3, scale, lin_wT)


_SIG = (2, 0, 1)


def _head_epilogue(a10, a01, a11, apos, w10_ref, w01_ref, w11_ref, out_ref):
    def wrow(ref, h):
        return ref[0:1, h:h + 1]

    def rows(acc, w_ref, d):
        return [sum(wrow(w_ref, h) * acc[d * h + k:d * h + k + 1, :]
                    for h in range(8)) for k in range(d)]

    h10 = rows(a10, w10_ref, 3)
    h01 = rows(a01, w01_ref, 3)
    h11 = rows(a11, w11_ref, 9)

    rvec = [h11[3 * _SIG[i] + _SIG[j]] for i in range(3) for j in range(3)]
    for k in range(9):
        out_ref[12 + k:13 + k, :] = rvec[k]

    norm2 = sum(r * r for r in rvec)
    norm = jnp.maximum(jnp.sqrt(norm2), 1e-5)
    rv = [r / norm for r in rvec]
    M = [[rv[3 * d + c] for d in range(3)] for c in range(3)]

    S = {}
    for i in range(3):
        for j in range(i, 3):
            S[(i, j)] = sum(M[c][i] * M[c][j] for c in range(3))
    one = jnp.ones_like(S[(0, 0)])
    zero = jnp.zeros_like(one)
    V = [[one if i == j else zero for j in range(3)] for i in range(3)]

    def sget(i, j):
        return S[(i, j)] if i <= j else S[(j, i)]

    def sset(i, j, v):
        S[(min(i, j), max(i, j))] = v

    for _ in range(6):
        for (p, q) in ((0, 1), (0, 2), (1, 2)):
            app, aqq, apq = sget(p, p), sget(q, q), sget(p, q)
            small = jnp.abs(apq) < 1e-30
            apq_s = jnp.where(small, one, apq)
            tau = (aqq - app) / (2.0 * apq_s)
            sgn = jnp.where(tau >= 0, one, -one)
            t = sgn / (jnp.abs(tau) + jnp.sqrt(1.0 + tau * tau))
            t = jnp.where(small, zero, t)
            c = jax.lax.rsqrt(1.0 + t * t)
            s = t * c
            r = 3 - p - q
            spr, sqr = sget(p, r), sget(q, r)
            sset(p, r, c * spr - s * sqr)
            sset(q, r, s * spr + c * sqr)
            sset(p, p, app - t * apq)
            sset(q, q, aqq + t * apq)
            sset(p, q, zero)
            for i in range(3):
                vip, viq = V[i][p], V[i][q]
                V[i][p] = c * vip - s * viq
                V[i][q] = s * vip + c * viq

    d = [sget(0, 0), sget(1, 1), sget(2, 2)]
    for (a, bcol) in ((0, 1), (0, 2), (1, 2)):
        sw = d[a] < d[bcol]
        d[a], d[bcol] = (jnp.where(sw, d[bcol], d[a]),
                         jnp.where(sw, d[a], d[bcol]))
        for i in range(3):
            va, vb = V[i][a], V[i][bcol]
            V[i][a] = jnp.where(sw, vb, va)
            V[i][bcol] = jnp.where(sw, va, vb)

    def matvec(col):
        return [sum(M[c][k] * V[k][col] for k in range(3)) for c in range(3)]

    u1 = matvec(0)
    n1 = jnp.sqrt(sum(u * u for u in u1))
    u1 = [u / jnp.maximum(n1, 1e-20) for u in u1]
    u2 = matvec(1)
    proj = sum(a_ * b_ for a_, b_ in zip(u1, u2))
    u2 = [u - proj * v for u, v in zip(u2, u1)]
    n2 = jnp.sqrt(sum(u * u for u in u2))
    u2 = [u / jnp.maximum(n2, 1e-20) for u in u2]
    u3 = [u1[1] * u2[2] - u1[2] * u2[1],
          u1[2] * u2[0] - u1[0] * u2[2],
          u1[0] * u2[1] - u1[1] * u2[0]]
    detV = (V[0][0] * (V[1][1] * V[2][2] - V[1][2] * V[2][1])
            - V[0][1] * (V[1][0] * V[2][2] - V[1][2] * V[2][0])
            + V[0][2] * (V[1][0] * V[2][1] - V[1][1] * V[2][0]))

    Rm = [[u1[c] * V[dd][0] + u2[c] * V[dd][1] + detV * u3[c] * V[dd][2]
           for dd in range(3)] for c in range(3)]
    for c in range(3):
        for dd in range(3):
            out_ref[3 * c + dd:3 * c + dd + 1, :] = Rm[c][dd]

    m1 = [apos[k:k + 1, :] for k in range(3)]
    m2 = [apos[3 + k:4 + k, :] for k in range(3)]
    tb = [h01[_SIG[i]] for i in range(3)]
    ta = [h10[_SIG[i]] for i in range(3)]
    for c in range(3):
        tv = m2[c] + tb[c] - sum(Rm[c][dd] * (m1[dd] + ta[dd])
                                 for dd in range(3))
        out_ref[9 + c:10 + c, :] = tv


def _stage3(x10f, x01f, x11f, pos, e, ids3, s, w10, w01, w11, K):
    def body(x10_ref, x01_ref, x11_ref, pos_ref, e_ref, ids_ref, s_ref,
             w10_ref, w01_ref, w11_ref, out_ref,
             a10_ref, a01_ref, a11_ref, apos_ref):
        i = pl.program_id(0)
        oh = _onehot(ids_ref[0])
        sg = _dot(oh, s_ref[...], 1, 0)
        W = e_ref[...] / sg[:, 0:3]
        w0, w1, w2 = W[:, 0:1], W[:, 1:2], W[:, 2:3]
        p8 = jnp.concatenate([pos_ref[...] * w2, jnp.zeros((R, 2), F32)],
                             axis=1)
        b10 = _dot(x10_ref[...] * w1, oh, 0, 0)
        b01 = _dot(x01_ref[...] * w1, oh, 0, 0)
        b11 = _dot(x11_ref[...] * w0, oh, 0, 0)
        bpos = _dot(p8, oh, 0, 0)

        @pl.when(i == 0)
        def _():
            a10_ref[...] = b10
            a01_ref[...] = b01
            a11_ref[...] = b11
            apos_ref[...] = bpos

        @pl.when(i > 0)
        def _():
            a10_ref[...] += b10
            a01_ref[...] += b01
            a11_ref[...] += b11
            apos_ref[...] += bpos

        @pl.when(i == K - 1)
        def _():
            _head_epilogue(a10_ref[...], a01_ref[...], a11_ref[...],
                           apos_ref[...], w10_ref, w01_ref, w11_ref, out_ref)

    return pl.pallas_call(
        body,
        grid=(K,),
        in_specs=[
            pl.BlockSpec((R, 24), lambda i: (i, 0)),
            pl.BlockSpec((R, 24), lambda i: (i, 0)),
            pl.BlockSpec((R, 72), lambda i: (i, 0)),
            pl.BlockSpec((R, 6), lambda i: (i, 0)),
            pl.BlockSpec((R, 3), lambda i: (i, 0)),
            pl.BlockSpec((1, R, 1), lambda i: (i, 0, 0)),
            pl.BlockSpec((B, 8), lambda i: (0, 0)),
            pl.BlockSpec((1, 8), lambda i: (0, 0)),
            pl.BlockSpec((1, 8), lambda i: (0, 0)),
            pl.BlockSpec((1, 8), lambda i: (0, 0)),
        ],
        out_specs=pl.BlockSpec((32, B), lambda i: (0, 0)),
        out_shape=jax.ShapeDtypeStruct((32, B), F32),
        scratch_shapes=[
            pltpu.VMEM((24, B), F32),
            pltpu.VMEM((24, B), F32),
            pltpu.VMEM((72, B), F32),
            pltpu.VMEM((8, B), F32),
        ],
    )(x10f, x01f, x11f, pos, e, ids3, s, w10, w01, w11)


def kernel(x00, x10, x01, x11, pos, segment_ids, gn_gamma, gn_beta,
           gn_alpha, lin_w, lin_b, W10, W01, W11):
    del gn_beta, lin_b
    N = x00.shape[0]
    assert N % R == 0
    K = N // R

    x2d = x00[:, :, 0]
    ids3 = segment_ids.astype(jnp.int32).reshape(K, R, 1)
    alpha_r = gn_alpha.reshape(1, 32)
    gamma_r = gn_gamma.reshape(1, 32)
    lin_wT = lin_w.T
    x10f = x10.reshape(N, 24)
    x01f = x01.reshape(N, 24)
    x11f = x11.reshape(N, 72)
    w10 = W10.reshape(1, 8)
    w01 = W01.reshape(1, 8)
    w11 = W11.reshape(1, 8)

    scale = _stage1(x2d, ids3, alpha_r, gamma_r, K)
    e, s = _stage2(x2d, ids3, scale, lin_wT, K, N)
    out32 = _stage3(x10f, x01f, x11f, pos, e, ids3, s, w10, w01, w11, K)

    rot = out32[0:9].T.reshape(B, 3, 3)
    t = out32[9:12].T
    r_vector = out32[12:21].T
    return rot, t, r_vector


# speedup vs baseline: 3.0947x; 3.0947x over previous
import jax
import jax.numpy as jnp
from jax.experimental import pallas as pl
from jax.experimental.pallas import tpu as pltpu

F32 = jnp.float32
B = 256
R = 2048
EPS = 1e-5

_SIG = (2, 0, 1)


def _dot(a, b, ca, cb):
    return jax.lax.dot_general(a, b, (((ca,), (cb,)), ((), ())),
                               preferred_element_type=F32)


def _onehot(ids):
    return (jax.lax.broadcasted_iota(jnp.int32, (B, ids.shape[1]), 0)
            == ids).astype(F32)


def _head_epilogue(a10, a01, a11, apos, s, w10_ref, w01_ref, w11_ref,
                   out_ref):
    s0 = jnp.maximum(s[0:1, :], 1e-30)
    s1 = jnp.maximum(s[1:2, :], 1e-30)
    s2 = jnp.maximum(s[2:3, :], 1e-30)

    def wrow(ref, h):
        return ref[0:1, h:h + 1]

    def rows(acc, w_ref, d, sden):
        return [sum(wrow(w_ref, h) * acc[d * h + k:d * h + k + 1, :]
                    for h in range(8)) / sden for k in range(d)]

    h10 = rows(a10, w10_ref, 3, s1)
    h01 = rows(a01, w01_ref, 3, s1)
    h11 = rows(a11, w11_ref, 9, s0)

    rvec = [h11[3 * _SIG[i] + _SIG[j]] for i in range(3) for j in range(3)]
    for k in range(9):
        out_ref[12 + k:13 + k, :] = rvec[k]

    norm2 = sum(r * r for r in rvec)
    norm = jnp.maximum(jnp.sqrt(norm2), 1e-5)
    rv = [r / norm for r in rvec]
    M = [[rv[3 * d + c] for d in range(3)] for c in range(3)]

    S = {}
    for i in range(3):
        for j in range(i, 3):
            S[(i, j)] = sum(M[c][i] * M[c][j] for c in range(3))
    one = jnp.ones_like(S[(0, 0)])
    zero = jnp.zeros_like(one)
    V = [[one if i == j else zero for j in range(3)] for i in range(3)]

    def sget(i, j):
        return S[(i, j)] if i <= j else S[(j, i)]

    def sset(i, j, v):
        S[(min(i, j), max(i, j))] = v

    for _ in range(6):
        for (p, q) in ((0, 1), (0, 2), (1, 2)):
            app, aqq, apq = sget(p, p), sget(q, q), sget(p, q)
            small = jnp.abs(apq) < 1e-30
            apq_s = jnp.where(small, one, apq)
            tau = (aqq - app) / (2.0 * apq_s)
            sgn = jnp.where(tau >= 0, one, -one)
            t = sgn / (jnp.abs(tau) + jnp.sqrt(1.0 + tau * tau))
            t = jnp.where(small, zero, t)
            c = jax.lax.rsqrt(1.0 + t * t)
            s_ = t * c
            r = 3 - p - q
            spr, sqr = sget(p, r), sget(q, r)
            sset(p, r, c * spr - s_ * sqr)
            sset(q, r, s_ * spr + c * sqr)
            sset(p, p, app - t * apq)
            sset(q, q, aqq + t * apq)
            sset(p, q, zero)
            for i in range(3):
                vip, viq = V[i][p], V[i][q]
                V[i][p] = c * vip - s_ * viq
                V[i][q] = s_ * vip + c * viq

    d = [sget(0, 0), sget(1, 1), sget(2, 2)]
    for (a, bcol) in ((0, 1), (0, 2), (1, 2)):
        sw = d[a] < d[bcol]
        d[a], d[bcol] = (jnp.where(sw, d[bcol], d[a]),
                         jnp.where(sw, d[a], d[bcol]))
        for i in range(3):
            va, vb = V[i][a], V[i][bcol]
            V[i][a] = jnp.where(sw, vb, va)
            V[i][bcol] = jnp.where(sw, va, vb)

    def matvec(col):
        return [sum(M[c][k] * V[k][col] for k in range(3)) for c in range(3)]

    u1 = matvec(0)
    n1 = jnp.sqrt(sum(u * u for u in u1))
    u1 = [u / jnp.maximum(n1, 1e-20) for u in u1]
    u2 = matvec(1)
    proj = sum(a_ * b_ for a_, b_ in zip(u1, u2))
    u2 = [u - proj * v for u, v in zip(u2, u1)]
    n2 = jnp.sqrt(sum(u * u for u in u2))
    u2 = [u / jnp.maximum(n2, 1e-20) for u in u2]
    u3 = [u1[1] * u2[2] - u1[2] * u2[1],
          u1[2] * u2[0] - u1[0] * u2[2],
          u1[0] * u2[1] - u1[1] * u2[0]]
    detV = (V[0][0] * (V[1][1] * V[2][2] - V[1][2] * V[2][1])
            - V[0][1] * (V[1][0] * V[2][2] - V[1][2] * V[2][0])
            + V[0][2] * (V[1][0] * V[2][1] - V[1][1] * V[2][0]))

    Rm = [[u1[c] * V[dd][0] + u2[c] * V[dd][1] + detV * u3[c] * V[dd][2]
           for dd in range(3)] for c in range(3)]
    for c in range(3):
        for dd in range(3):
            out_ref[3 * c + dd:3 * c + dd + 1, :] = Rm[c][dd]

    m1 = [apos[k:k + 1, :] / s2 for k in range(3)]
    m2 = [apos[3 + k:4 + k, :] / s2 for k in range(3)]
    tb = [h01[_SIG[i]] for i in range(3)]
    ta = [h10[_SIG[i]] for i in range(3)]
    for c in range(3):
        tv = m2[c] + tb[c] - sum(Rm[c][dd] * (m1[dd] + ta[dd])
                                 for dd in range(3))
        out_ref[9 + c:10 + c, :] = tv


def _fused(xt, ids3, x10t, x01t, x11t, post, alpha_c, gamma_c, lin_w,
           w10, w01, w11, K):
    def body(x_ref, ids_ref, x10_ref, x01_ref, x11_ref, pos_ref,
             alpha_ref, gamma_ref, w_ref, w10_ref, w01_ref, w11_ref,
             out_ref, xc_ref, stat_ref, scale_ref, s_ref,
             a10_ref, a01_ref, a11_ref, apos_ref):
        i = pl.program_id(0)
        oh = _onehot(ids_ref[0])

        @pl.when(i < K)
        def _():
            x = x_ref[...]
            xc_ref[pl.ds(i * 32, 32), :] = x
            vals = jnp.concatenate([x, x * x, jnp.ones((8, R), F32)],
                                   axis=0)
            blk = _dot(vals, oh, 1, 1)

            @pl.when(i == 0)
            def _():
                stat_ref[...] = blk

            @pl.when(i > 0)
            def _():
                stat_ref[...] += blk

            @pl.when(i == K - 1)
            def _():
                acc = stat_ref[...]
                inv = 1.0 / jnp.maximum(acc[64:65, :], 1.0)
                mean = acc[0:32, :] * inv
                ex2 = acc[32:64, :] * inv
                a = alpha_ref[...]
                var = ex2 - (2.0 * a - a * a) * mean * mean
                scale_ref[...] = gamma_ref[...] * jax.lax.rsqrt(var + EPS)

        @pl.when(i >= K)
        def _():
            j = i - K
            x = xc_ref[pl.ds(j * 32, 32), :]
            sc = _dot(scale_ref[...], oh, 1, 0)
            logits = _dot(w_ref[...], x * sc, 1, 0)
            eb = jnp.exp(logits)
            e8 = jnp.concatenate([eb, jnp.zeros((5, R), F32)], axis=0)
            bs = _dot(e8, oh, 1, 1)
            b10 = _dot(x10_ref[...] * eb[1:2, :], oh, 1, 1)
            b01 = _dot(x01_ref[...] * eb[1:2, :], oh, 1, 1)
            b11 = _dot(x11_ref[...] * eb[0:1, :], oh, 1, 1)
            p8 = jnp.concatenate([pos_ref[...] * eb[2:3, :],
                                  jnp.zeros((2, R), F32)], axis=0)
            bpos = _dot(p8, oh, 1, 1)

            @pl.when(j == 0)
            def _():
                s_ref[...] = bs
                a10_ref[...] = b10
                a01_ref[...] = b01
                a11_ref[...] = b11
                apos_ref[...] = bpos

            @pl.when(j > 0)
            def _():
                s_ref[...] += bs
                a10_ref[...] += b10
                a01_ref[...] += b01
                a11_ref[...] += b11
                apos_ref[...] += bpos

            @pl.when(j == K - 1)
            def _():
                _head_epilogue(a10_ref[...], a01_ref[...], a11_ref[...],
                               apos_ref[...], s_ref[...],
                               w10_ref, w01_ref, w11_ref, out_ref)

    full = lambda i: (0, 0)
    return pl.pallas_call(
        body,
        grid=(2 * K,),
        in_specs=[
            pl.BlockSpec((32, R), lambda i: (0, jnp.minimum(i, K - 1))),
            pl.BlockSpec((1, 1, R),
                         lambda i: (jnp.where(i < K, i, i - K), 0, 0)),
            pl.BlockSpec((24, R), lambda i: (0, jnp.maximum(i - K, 0))),
            pl.BlockSpec((24, R), lambda i: (0, jnp.maximum(i - K, 0))),
            pl.BlockSpec((72, R), lambda i: (0, jnp.maximum(i - K, 0))),
            pl.BlockSpec((6, R), lambda i: (0, jnp.maximum(i - K, 0))),
            pl.BlockSpec((32, 1), full),
            pl.BlockSpec((32, 1), full),
            pl.BlockSpec((3, 32), full),
            pl.BlockSpec((1, 8), full),
            pl.BlockSpec((1, 8), full),
            pl.BlockSpec((1, 8), full),
        ],
        out_specs=pl.BlockSpec((32, B), full),
        out_shape=jax.ShapeDtypeStruct((32, B), F32),
        scratch_shapes=[
            pltpu.VMEM((K * 32, R), F32),
            pltpu.VMEM((72, B), F32),
            pltpu.VMEM((32, B), F32),
            pltpu.VMEM((8, B), F32),
            pltpu.VMEM((24, B), F32),
            pltpu.VMEM((24, B), F32),
            pltpu.VMEM((72, B), F32),
            pltpu.VMEM((8, B), F32),
        ],
    )(xt, ids3, x10t, x01t, x11t, post, alpha_c, gamma_c, lin_w,
      w10, w01, w11)


def kernel(x00, x10, x01, x11, pos, segment_ids, gn_gamma, gn_beta,
           gn_alpha, lin_w, lin_b, W10, W01, W11):
    del gn_beta, lin_b
    N = x00.shape[0]
    NP = -(-N // R) * R
    K = NP // R
    P = NP - N

    def padt(a):
        return jnp.pad(a, ((0, 0), (0, P))) if P else a

    xt = padt(x00[:, :, 0].T)
    ids_p = jnp.pad(segment_ids.astype(jnp.int32), (0, P),
                    constant_values=B)
    ids3 = ids_p.reshape(K, 1, R)
    alpha_c = gn_alpha.reshape(32, 1)
    gamma_c = gn_gamma.reshape(32, 1)
    x10t = padt(x10.reshape(N, 24).T)
    x01t = padt(x01.reshape(N, 24).T)
    x11t = padt(x11.reshape(N, 72).T)
    post = padt(pos.T)
    w10 = W10.reshape(1, 8)
    w01 = W01.reshape(1, 8)
    w11 = W11.reshape(1, 8)

    out32 = _fused(xt, ids3, x10t, x01t, x11t, post, alpha_c, gamma_c,
                   lin_w, w10, w01, w11, K)

    rot = out32[0:9].T.reshape(B, 3, 3)
    t = out32[9:12].T
    r_vector = out32[12:21].T
    return rot, t, r_vector


# speedup vs baseline: 3.5082x; 1.1336x over previous
import jax
import jax.numpy as jnp
from jax.experimental import pallas as pl
from jax.experimental.pallas import tpu as pltpu

F32 = jnp.float32
B = 256
R = 2048
EPS = 1e-5

_SIG = (2, 0, 1)


def _dot(a, b, ca, cb):
    return jax.lax.dot_general(a, b, (((ca,), (cb,)), ((), ())),
                               preferred_element_type=F32)


def _onehot(ids):
    return (jax.lax.broadcasted_iota(jnp.int32, (B, ids.shape[1]), 0)
            == ids).astype(F32)


def _head_epilogue(a10, a01, a11, apos, s, w10_ref, w01_ref, w11_ref,
                   out_ref):
    s0 = jnp.maximum(s[0:1, :], 1e-30)
    s1 = jnp.maximum(s[1:2, :], 1e-30)
    s2 = jnp.maximum(s[2:3, :], 1e-30)

    def wrow(ref, h):
        return ref[0:1, h:h + 1]

    def rows(acc, w_ref, d, sden):
        return [sum(wrow(w_ref, h) * acc[d * h + k:d * h + k + 1, :]
                    for h in range(8)) / sden for k in range(d)]

    h10 = rows(a10, w10_ref, 3, s1)
    h01 = rows(a01, w01_ref, 3, s1)
    h11 = rows(a11, w11_ref, 9, s0)

    rvec = [h11[3 * _SIG[i] + _SIG[j]] for i in range(3) for j in range(3)]
    for k in range(9):
        out_ref[12 + k:13 + k, :] = rvec[k]

    norm2 = sum(r * r for r in rvec)
    norm = jnp.maximum(jnp.sqrt(norm2), 1e-5)
    rv = [r / norm for r in rvec]
    M = [[rv[3 * d + c] for d in range(3)] for c in range(3)]

    S = {}
    for i in range(3):
        for j in range(i, 3):
            S[(i, j)] = sum(M[c][i] * M[c][j] for c in range(3))
    one = jnp.ones_like(S[(0, 0)])
    zero = jnp.zeros_like(one)
    V = [[one if i == j else zero for j in range(3)] for i in range(3)]

    def sget(i, j):
        return S[(i, j)] if i <= j else S[(j, i)]

    def sset(i, j, v):
        S[(min(i, j), max(i, j))] = v

    for _ in range(6):
        for (p, q) in ((0, 1), (0, 2), (1, 2)):
            app, aqq, apq = sget(p, p), sget(q, q), sget(p, q)
            small = jnp.abs(apq) < 1e-30
            apq_s = jnp.where(small, one, apq)
            tau = (aqq - app) / (2.0 * apq_s)
            sgn = jnp.where(tau >= 0, one, -one)
            t = sgn / (jnp.abs(tau) + jnp.sqrt(1.0 + tau * tau))
            t = jnp.where(small, zero, t)
            c = jax.lax.rsqrt(1.0 + t * t)
            s_ = t * c
            r = 3 - p - q
            spr, sqr = sget(p, r), sget(q, r)
            sset(p, r, c * spr - s_ * sqr)
            sset(q, r, s_ * spr + c * sqr)
            sset(p, p, app - t * apq)
            sset(q, q, aqq + t * apq)
            sset(p, q, zero)
            for i in range(3):
                vip, viq = V[i][p], V[i][q]
                V[i][p] = c * vip - s_ * viq
                V[i][q] = s_ * vip + c * viq

    d = [sget(0, 0), sget(1, 1), sget(2, 2)]
    for (a, bcol) in ((0, 1), (0, 2), (1, 2)):
        sw = d[a] < d[bcol]
        d[a], d[bcol] = (jnp.where(sw, d[bcol], d[a]),
                         jnp.where(sw, d[a], d[bcol]))
        for i in range(3):
            va, vb = V[i][a], V[i][bcol]
            V[i][a] = jnp.where(sw, vb, va)
            V[i][bcol] = jnp.where(sw, va, vb)

    def matvec(col):
        return [sum(M[c][k] * V[k][col] for k in range(3)) for c in range(3)]

    u1 = matvec(0)
    n1 = jnp.sqrt(sum(u * u for u in u1))
    u1 = [u / jnp.maximum(n1, 1e-20) for u in u1]
    u2 = matvec(1)
    proj = sum(a_ * b_ for a_, b_ in zip(u1, u2))
    u2 = [u - proj * v for u, v in zip(u2, u1)]
    n2 = jnp.sqrt(sum(u * u for u in u2))
    u2 = [u / jnp.maximum(n2, 1e-20) for u in u2]
    u3 = [u1[1] * u2[2] - u1[2] * u2[1],
          u1[2] * u2[0] - u1[0] * u2[2],
          u1[0] * u2[1] - u1[1] * u2[0]]
    detV = (V[0][0] * (V[1][1] * V[2][2] - V[1][2] * V[2][1])
            - V[0][1] * (V[1][0] * V[2][2] - V[1][2] * V[2][0])
            + V[0][2] * (V[1][0] * V[2][1] - V[1][1] * V[2][0]))

    Rm = [[u1[c] * V[dd][0] + u2[c] * V[dd][1] + detV * u3[c] * V[dd][2]
           for dd in range(3)] for c in range(3)]
    for c in range(3):
        for dd in range(3):
            out_ref[3 * c + dd:3 * c + dd + 1, :] = Rm[c][dd]

    m1 = [apos[k:k + 1, :] / s2 for k in range(3)]
    m2 = [apos[3 + k:4 + k, :] / s2 for k in range(3)]
    tb = [h01[_SIG[i]] for i in range(3)]
    ta = [h10[_SIG[i]] for i in range(3)]
    for c in range(3):
        tv = m2[c] + tb[c] - sum(Rm[c][dd] * (m1[dd] + ta[dd])
                                 for dd in range(3))
        out_ref[9 + c:10 + c, :] = tv


def _stats(xt, ids3, alpha_c, gamma_c, K):
    def body(x_ref, ids_ref, alpha_ref, gamma_ref, scale_ref, stat_ref):
        i = pl.program_id(0)
        oh = _onehot(ids_ref[0])
        x = x_ref[...]
        vals = jnp.concatenate([x, x * x, jnp.ones((8, R), F32)], axis=0)
        blk = _dot(vals, oh, 1, 1)

        @pl.when(i == 0)
        def _():
            stat_ref[...] = blk

        @pl.when(i > 0)
        def _():
            stat_ref[...] += blk

        @pl.when(i == K - 1)
        def _():
            acc = stat_ref[...]
            inv = 1.0 / jnp.maximum(acc[64:65, :], 1.0)
            mean = acc[0:32, :] * inv
            ex2 = acc[32:64, :] * inv
            a = alpha_ref[...]
            var = ex2 - (2.0 * a - a * a) * mean * mean
            scale_ref[...] = gamma_ref[...] * jax.lax.rsqrt(var + EPS)

    return pl.pallas_call(
        body,
        grid=(K,),
        in_specs=[
            pl.BlockSpec((32, R), lambda i: (0, i)),
            pl.BlockSpec((1, 1, R), lambda i: (i, 0, 0)),
            pl.BlockSpec((32, 1), lambda i: (0, 0)),
            pl.BlockSpec((32, 1), lambda i: (0, 0)),
        ],
        out_specs=pl.BlockSpec((32, B), lambda i: (0, 0)),
        out_shape=jax.ShapeDtypeStruct((32, B), F32),
        scratch_shapes=[pltpu.VMEM((72, B), F32)],
    )(xt, ids3, alpha_c, gamma_c)


def _main(xt, ids3, x10t, x01t, x11t, post, scale, lin_w, w10, w01, w11, K):
    def body(x_ref, ids_ref, x10_ref, x01_ref, x11_ref, pos_ref,
             scale_ref, w_ref, w10_ref, w01_ref, w11_ref,
             out_ref, s_ref, a10_ref, a01_ref, a11_ref, apos_ref):
        j = pl.program_id(0)
        oh = _onehot(ids_ref[0])
        sc = _dot(scale_ref[...], oh, 1, 0)
        logits = _dot(w_ref[...], x_ref[...] * sc, 1, 0)
        eb = jnp.exp(logits)
        e8 = jnp.concatenate([eb, jnp.zeros((5, R), F32)], axis=0)
        bs = _dot(e8, oh, 1, 1)
        b10 = _dot(x10_ref[...] * eb[1:2, :], oh, 1, 1)
        b01 = _dot(x01_ref[...] * eb[1:2, :], oh, 1, 1)
        b11 = _dot(x11_ref[...] * eb[0:1, :], oh, 1, 1)
        p8 = jnp.concatenate([pos_ref[...] * eb[2:3, :],
                              jnp.zeros((2, R), F32)], axis=0)
        bpos = _dot(p8, oh, 1, 1)

        @pl.when(j == 0)
        def _():
            s_ref[...] = bs
            a10_ref[...] = b10
            a01_ref[...] = b01
            a11_ref[...] = b11
            apos_ref[...] = bpos

        @pl.when(j > 0)
        def _():
            s_ref[...] += bs
            a10_ref[...] += b10
            a01_ref[...] += b01
            a11_ref[...] += b11
            apos_ref[...] += bpos

        @pl.when(j == K - 1)
        def _():
            _head_epilogue(a10_ref[...], a01_ref[...], a11_ref[...],
                           apos_ref[...], s_ref[...],
                           w10_ref, w01_ref, w11_ref, out_ref)

    full = lambda i: (0, 0)
    return pl.pallas_call(
        body,
        grid=(K,),
        in_specs=[
            pl.BlockSpec((32, R), lambda i: (0, i)),
            pl.BlockSpec((1, 1, R), lambda i: (i, 0, 0)),
            pl.BlockSpec((24, R), lambda i: (0, i)),
            pl.BlockSpec((24, R), lambda i: (0, i)),
            pl.BlockSpec((72, R), lambda i: (0, i)),
            pl.BlockSpec((6, R), lambda i: (0, i)),
            pl.BlockSpec((32, B), full),
            pl.BlockSpec((3, 32), full),
            pl.BlockSpec((1, 8), full),
            pl.BlockSpec((1, 8), full),
            pl.BlockSpec((1, 8), full),
        ],
        out_specs=pl.BlockSpec((32, B), full),
        out_shape=jax.ShapeDtypeStruct((32, B), F32),
        scratch_shapes=[
            pltpu.VMEM((8, B), F32),
            pltpu.VMEM((24, B), F32),
            pltpu.VMEM((24, B), F32),
            pltpu.VMEM((72, B), F32),
            pltpu.VMEM((8, B), F32),
        ],
    )(xt, ids3, x10t, x01t, x11t, post, scale, lin_w, w10, w01, w11)


def kernel(x00, x10, x01, x11, pos, segment_ids, gn_gamma, gn_beta,
           gn_alpha, lin_w, lin_b, W10, W01, W11):
    del gn_beta, lin_b
    N = x00.shape[0]
    NP = -(-N // R) * R
    K = NP // R
    P = NP - N

    def padt(a):
        return jnp.pad(a, ((0, 0), (0, P))) if P else a

    xt = padt(x00[:, :, 0].T)
    ids_p = jnp.pad(segment_ids.astype(jnp.int32), (0, P),
                    constant_values=B)
    ids3 = ids_p.reshape(K, 1, R)
    alpha_c = gn_alpha.reshape(32, 1)
    gamma_c = gn_gamma.reshape(32, 1)
    x10t = padt(x10.reshape(N, 24).T)
    x01t = padt(x01.reshape(N, 24).T)
    x11t = padt(x11.reshape(N, 72).T)
    post = padt(pos.T)
    w10 = W10.reshape(1, 8)
    w01 = W01.reshape(1, 8)
    w11 = W11.reshape(1, 8)

    scale = _stats(xt, ids3, alpha_c, gamma_c, K)
    out32 = _main(xt, ids3, x10t, x01t, x11t, post, scale, lin_w,
                  w10, w01, w11, K)

    rot = out32[0:9].T.reshape(B, 3, 3)
    t = out32[9:12].T
    r_vector = out32[12:21].T
    return rot, t, r_vector
